# baseline (speedup 1.0000x reference)
import functools

import jax
import jax.numpy as jnp
from jax import lax
from jax.experimental import pallas as pl
from jax.experimental.pallas import tpu as pltpu
from jax.experimental.pallas import tpu_sc as plsc

N = 10000
D = 256
G = 16
NP = 10240
E = 160000
EROWS = 1280
E2 = EROWS * 128
NC = 2
NS = 16
RPS = EROWS // NS
NROWS_PS = NP // NS
DH = D // NC

_f32 = jnp.float32


def _sc_agg_body(y2, srcp, dst2, zrows, agg, src_v, dst_v, rows_v, sem, acc):
    c = lax.axis_index("c")
    s = lax.axis_index("s")
    row0 = s * NROWS_PS
    pltpu.sync_copy(zrows, acc.at[pl.ds(row0, NROWS_PS)])
    pltpu.sync_copy(srcp.at[c, pl.ds(s * RPS, RPS), :], src_v)
    pltpu.sync_copy(dst2.at[pl.ds(s * RPS, RPS), :], dst_v)
    plsc.subcore_barrier()

    def body(j, carry):
        pltpu.async_copy(y2.at[src_v.at[j]], rows_v, sem).wait()
        pltpu.sync_copy(rows_v, acc.at[dst_v.at[j]], add=True)
        return carry

    lax.fori_loop(0, RPS, body, 0)
    plsc.subcore_barrier()
    pltpu.sync_copy(acc.at[pl.ds(row0, NROWS_PS)],
                    agg.at[pl.ds(c * NP + row0, NROWS_PS)])


@functools.cache
def _get_sc_agg():
    return pl.kernel(
        _sc_agg_body,
        out_type=jax.ShapeDtypeStruct((NC * NP, DH), _f32),
        mesh=plsc.VectorSubcoreMesh(
            core_axis_name="c", subcore_axis_name="s",
            num_cores=NC, num_subcores=NS),
        scratch_types=[
            pltpu.VMEM((RPS, 128), jnp.int32),
            pltpu.VMEM((RPS, 128), jnp.int32),
            pltpu.VMEM((128, DH), _f32),
            pltpu.SemaphoreType.DMA,
            pltpu.VMEM_SHARED((NP, DH), _f32),
        ],
    )


def _sc_agg(y2, srcp, dst2, zrows):
    return _get_sc_agg()(y2, srcp, dst2, zrows)


def _sc_deg_body(dst2, ones_tab, zrows, deg, dst_v, ones_v, acc):
    c = lax.axis_index("c")
    s = lax.axis_index("s")
    row0 = s * NROWS_PS
    pltpu.sync_copy(zrows, acc.at[pl.ds(row0, NROWS_PS)])
    pltpu.sync_copy(ones_tab, ones_v)
    pltpu.sync_copy(dst2.at[pl.ds(s * RPS, RPS), :], dst_v)
    plsc.subcore_barrier()

    def body(j, carry):
        pltpu.sync_copy(ones_v, acc.at[dst_v.at[j]], add=True)
        return carry
    lax.fori_loop(0, RPS, body, 0)

    plsc.subcore_barrier()
    pltpu.sync_copy(acc.at[pl.ds(row0, NROWS_PS)],
                    deg.at[pl.ds(c * NP + row0, NROWS_PS)])


@functools.cache
def _get_sc_deg():
    return pl.kernel(
        _sc_deg_body,
        out_type=jax.ShapeDtypeStruct((NC * NP, 128), _f32),
        mesh=plsc.VectorSubcoreMesh(
            core_axis_name="c", subcore_axis_name="s",
            num_cores=NC, num_subcores=NS),
        scratch_types=[
            pltpu.VMEM((RPS, 128), jnp.int32),
            pltpu.VMEM((128, 128), _f32),
            pltpu.VMEM_SHARED((NP, 128), _f32),
        ],
    )


def _sc_deg(dst2, ones_tab, zrows):
    return _get_sc_deg()(dst2, ones_tab, zrows)


def _dinv_block(deg_ref):
    d = (deg_ref[0, :, 0:1] + deg_ref[1, :, 0:1]) * 0.5 + 1.0
    return lax.rsqrt(d)


def _first_body(deg, x, w, out, acc):
    k = pl.program_id(2)

    @pl.when(k == 0)
    def _():
        acc[...] = jnp.zeros_like(acc)

    acc[...] += jnp.dot(x[...], w[...], preferred_element_type=_f32)

    @pl.when(k == 1)
    def _():
        out[0] = acc[...] * _dinv_block(deg)


def _mid_body(deg, agg, y, w, b, out, acc):
    k = pl.program_id(2)
    dinv = _dinv_block(deg)
    h = jnp.maximum(dinv * (agg[0] + y[0]) + b[0], 0.0)

    @pl.when(k == 0)
    def _():
        acc[...] = jnp.zeros_like(acc)

    acc[...] += jnp.dot(h, w[...], preferred_element_type=_f32)

    @pl.when(k == 1)
    def _():
        out[0] = acc[...] * dinv


def _final_body(deg, agg, y, b, wa, ba, wp, bp, batch, out, p_acc, s_acc):
    i = pl.program_id(0)
    dinv = _dinv_block(deg)
    h0 = jnp.maximum(dinv * (agg[0] + y[0]) + b[0], 0.0)
    h1 = jnp.maximum(dinv * (agg[1] + y[1]) + b[1], 0.0)
    h = jnp.concatenate([h0, h1], axis=1)
    sc = jnp.sum(h * wa[...], axis=1, keepdims=True) + ba[...]
    e = jnp.exp(jnp.tanh(sc))
    rows = i * 128 + lax.broadcasted_iota(jnp.int32, (128, 1), 0)
    e = jnp.where(rows < N, e, 0.0)
    onehot = (lax.broadcasted_iota(jnp.int32, (G, 128), 0)
              == batch[0]).astype(_f32)

    @pl.when(i == 0)
    def _():
        p_acc[...] = jnp.zeros_like(p_acc)
        s_acc[0, 0] = 0.0

    p_acc[...] += jnp.dot(onehot, h * e, preferred_element_type=_f32)
    s_acc[0, 0] += jnp.sum(e)

    @pl.when(i == NP // 128 - 1)
    def _():
        pooled = p_acc[...] / s_acc[0, 0]
        o = jnp.dot(pooled, wp[...], preferred_element_type=_f32) + bp[...]
        nrm = jnp.sqrt(jnp.sum(o * o, axis=1, keepdims=True))
        out[...] = o / jnp.maximum(nrm, 1e-12)


def _tc_first(deg3, x_pad, w):
    return pl.pallas_call(
        _first_body,
        grid=(NP // 128, 2, 2),
        in_specs=[
            pl.BlockSpec((2, 128, 128), lambda i, j, k: (0, i, 0)),
            pl.BlockSpec((128, 128), lambda i, j, k: (i, k)),
            pl.BlockSpec((128, 128), lambda i, j, k: (k, j)),
        ],
        out_specs=pl.BlockSpec((1, 128, 128), lambda i, j, k: (j, i, 0)),
        out_shape=jax.ShapeDtypeStruct((2, NP, 128), _f32),
        scratch_shapes=[pltpu.VMEM((128, 128), _f32)],
    )(deg3, x_pad, w)


def _tc_mid(deg3, agg3, y3, w, b2):
    return pl.pallas_call(
        _mid_body,
        grid=(NP // 128, 2, 2),
        in_specs=[
            pl.BlockSpec((2, 128, 128), lambda i, j, k: (0, i, 0)),
            pl.BlockSpec((1, 128, 128), lambda i, j, k: (k, i, 0)),
            pl.BlockSpec((1, 128, 128), lambda i, j, k: (k, i, 0)),
            pl.BlockSpec((128, 128), lambda i, j, k: (k, j)),
            pl.BlockSpec((1, 1, 128), lambda i, j, k: (k, 0, 0)),
        ],
        out_specs=pl.BlockSpec((1, 128, 128), lambda i, j, k: (j, i, 0)),
        out_shape=jax.ShapeDtypeStruct((2, NP, 128), _f32),
        scratch_shapes=[pltpu.VMEM((128, 128), _f32)],
    )(deg3, agg3, y3, w, b2)


def _tc_final(deg3, agg3, y3, b2, wa_row, ba2, wp, bp2, batch2):
    return pl.pallas_call(
        _final_body,
        grid=(NP // 128,),
        in_specs=[
            pl.BlockSpec((2, 128, 128), lambda i: (0, i, 0)),
            pl.BlockSpec((2, 128, 128), lambda i: (0, i, 0)),
            pl.BlockSpec((2, 128, 128), lambda i: (0, i, 0)),
            pl.BlockSpec((2, 1, 128), lambda i: (0, 0, 0)),
            pl.BlockSpec((1, 256), lambda i: (0, 0)),
            pl.BlockSpec((1, 1), lambda i: (0, 0)),
            pl.BlockSpec((256, 256), lambda i: (0, 0)),
            pl.BlockSpec((1, 256), lambda i: (0, 0)),
            pl.BlockSpec((1, 1, 128), lambda i: (i, 0, 0)),
        ],
        out_specs=pl.BlockSpec((G, 256), lambda i: (0, 0)),
        out_shape=jax.ShapeDtypeStruct((G, 256), _f32),
        scratch_shapes=[pltpu.VMEM((G, 256), _f32),
                        pltpu.SMEM((1, 1), _f32)],
    )(deg3, agg3, y3, b2, wa_row, ba2, wp, bp2, batch2)


def kernel(x, edge_index, batch, W1, b1, W2, b2, W3, b3, Wa, ba, Wp, bp):
    src = edge_index[0].astype(jnp.int32)
    dst = edge_index[1].astype(jnp.int32)
    pad = jnp.full((E2 - E,), NP - 1, dtype=jnp.int32)
    src_p = jnp.concatenate([src, pad])
    dst_p = jnp.concatenate([dst, pad])
    srcp = jnp.stack([src_p, src_p + NP]).reshape(NC, EROWS, 128)
    dst2 = dst_p.reshape(EROWS, 128)

    x_pad = jnp.zeros((NP, D), _f32).at[:N].set(x)
    batch2 = jnp.zeros((NP,), jnp.int32).at[:N].set(
        batch.astype(jnp.int32)).reshape(NP // 128, 128)

    zrows = jnp.zeros((NROWS_PS, DH), _f32)
    ones_tab = jnp.ones((128, 128), _f32)

    deg = _sc_deg(dst2, ones_tab, zrows)
    deg3 = deg.reshape(NC, NP, 128)

    y1 = _tc_first(deg3, x_pad, W1)
    agg1 = _sc_agg(y1.reshape(NC * NP, DH), srcp, dst2, zrows)
    y2 = _tc_mid(deg3, agg1.reshape(NC, NP, DH), y1, W2, b1.reshape(2, 1, 128))
    agg2 = _sc_agg(y2.reshape(NC * NP, DH), srcp, dst2, zrows)
    y3 = _tc_mid(deg3, agg2.reshape(NC, NP, DH), y2, W3, b2.reshape(2, 1, 128))
    agg3 = _sc_agg(y3.reshape(NC * NP, DH), srcp, dst2, zrows)

    out = _tc_final(
        deg3, agg3.reshape(NC, NP, DH), y3, b3.reshape(2, 1, 128),
        Wa.reshape(1, D), ba.reshape(1, 1), Wp, bp.reshape(1, D),
        batch2.reshape(NP // 128, 1, 128))
    return out


# speedup vs baseline: 1.0821x; 1.0821x over previous
import functools

import jax
import jax.numpy as jnp
from jax import lax
from jax.experimental import pallas as pl
from jax.experimental.pallas import tpu as pltpu
from jax.experimental.pallas import tpu_sc as plsc

N = 10000
D = 256
G = 16
NP = 10240
E = 160000
EROWS = 1280
E2 = EROWS * 128
NC = 2
NS = 16
RPS = EROWS // NS
NROWS_PS = NP // NS
DH = D // NC

_f32 = jnp.float32


def _sc_agg_body(y2, srcp, dst2, zrows, agg,
                 src_v, dst_v, rows0, rows1, gs0, gs1, acc):
    c = lax.axis_index("c")
    s = lax.axis_index("s")
    row0 = s * NROWS_PS
    pltpu.sync_copy(zrows, acc.at[pl.ds(row0, NROWS_PS)])
    pltpu.sync_copy(dst2.at[pl.ds(s * RPS, RPS), :], dst_v)
    plsc.subcore_barrier()

    half_rows = RPS // 2
    for half in range(2):
        pltpu.sync_copy(
            srcp.at[c, pl.ds(s * RPS + half * half_rows, half_rows), :],
            src_v)
        d_base = half * half_rows

        def body(g, carry):
            j0 = 4 * g
            d0 = pltpu.async_copy(y2.at[src_v.at[j0]], rows0, gs0)
            d1 = pltpu.async_copy(y2.at[src_v.at[j0 + 1]], rows1, gs1)
            d0.wait()
            pltpu.sync_copy(rows0, acc.at[dst_v.at[d_base + j0]], add=True)
            d2 = pltpu.async_copy(y2.at[src_v.at[j0 + 2]], rows0, gs0)
            d1.wait()
            pltpu.sync_copy(rows1, acc.at[dst_v.at[d_base + j0 + 1]],
                            add=True)
            d3 = pltpu.async_copy(y2.at[src_v.at[j0 + 3]], rows1, gs1)
            d2.wait()
            pltpu.sync_copy(rows0, acc.at[dst_v.at[d_base + j0 + 2]],
                            add=True)
            d3.wait()
            pltpu.sync_copy(rows1, acc.at[dst_v.at[d_base + j0 + 3]],
                            add=True)
            return carry

        lax.fori_loop(0, half_rows // 4, body, 0)

    plsc.subcore_barrier()
    pltpu.sync_copy(acc.at[pl.ds(row0, NROWS_PS)],
                    agg.at[pl.ds(c * NP + row0, NROWS_PS)])


@functools.cache
def _get_sc_agg():
    return pl.kernel(
        _sc_agg_body,
        out_type=jax.ShapeDtypeStruct((NC * NP, DH), _f32),
        mesh=plsc.VectorSubcoreMesh(
            core_axis_name="c", subcore_axis_name="s",
            num_cores=NC, num_subcores=NS),
        scratch_types=[
            pltpu.VMEM((RPS // 2, 128), jnp.int32),
            pltpu.VMEM((RPS, 128), jnp.int32),
            pltpu.VMEM((128, DH), _f32),
            pltpu.VMEM((128, DH), _f32),
            pltpu.SemaphoreType.DMA,
            pltpu.SemaphoreType.DMA,
            pltpu.VMEM_SHARED((NP, DH), _f32),
        ],
    )


def _sc_agg(y2, srcp, dst2, zrows):
    return _get_sc_agg()(y2, srcp, dst2, zrows)


def _sc_deg_body(dst2, ones_tab, zrows, deg, dst_v, ones_v, acc):
    c = lax.axis_index("c")
    s = lax.axis_index("s")
    row0 = s * NROWS_PS
    pltpu.sync_copy(zrows, acc.at[pl.ds(row0, NROWS_PS)])
    pltpu.sync_copy(ones_tab, ones_v)
    pltpu.sync_copy(
        dst2.at[pl.ds((c * NS + s) * (RPS // 2), RPS // 2), :], dst_v)
    plsc.subcore_barrier()

    def body(j, carry):
        pltpu.sync_copy(ones_v, acc.at[dst_v.at[j]], add=True)
        return carry
    lax.fori_loop(0, RPS // 2, body, 0)

    plsc.subcore_barrier()
    pltpu.sync_copy(acc.at[pl.ds(row0, NROWS_PS)],
                    deg.at[pl.ds(c * NP + row0, NROWS_PS)])


@functools.cache
def _get_sc_deg():
    return pl.kernel(
        _sc_deg_body,
        out_type=jax.ShapeDtypeStruct((NC * NP, 128), _f32),
        mesh=plsc.VectorSubcoreMesh(
            core_axis_name="c", subcore_axis_name="s",
            num_cores=NC, num_subcores=NS),
        scratch_types=[
            pltpu.VMEM((RPS // 2, 128), jnp.int32),
            pltpu.VMEM((128, 128), _f32),
            pltpu.VMEM_SHARED((NP, 128), _f32),
        ],
    )


def _sc_deg(dst2, ones_tab, zrows):
    return _get_sc_deg()(dst2, ones_tab, zrows)


def _dinv_block(deg_ref):
    d = deg_ref[0, :, 0:1] + deg_ref[1, :, 0:1] + 1.0
    return lax.rsqrt(d)


def _first_body(deg, x, w, out, acc):
    k = pl.program_id(2)

    @pl.when(k == 0)
    def _():
        acc[...] = jnp.zeros_like(acc)

    acc[...] += jnp.dot(x[...], w[...], preferred_element_type=_f32)

    @pl.when(k == 1)
    def _():
        out[0] = acc[...] * _dinv_block(deg)


def _mid_body(deg, agg, y, w, b, out, acc):
    k = pl.program_id(2)
    dinv = _dinv_block(deg)
    h = jnp.maximum(dinv * (agg[0] + y[0]) + b[0], 0.0)

    @pl.when(k == 0)
    def _():
        acc[...] = jnp.zeros_like(acc)

    acc[...] += jnp.dot(h, w[...], preferred_element_type=_f32)

    @pl.when(k == 1)
    def _():
        out[0] = acc[...] * dinv


def _final_body(deg, agg, y, b, wa, ba, wp, bp, batch, out, p_acc, s_acc):
    i = pl.program_id(0)
    dinv = _dinv_block(deg)
    h0 = jnp.maximum(dinv * (agg[0] + y[0]) + b[0], 0.0)
    h1 = jnp.maximum(dinv * (agg[1] + y[1]) + b[1], 0.0)
    h = jnp.concatenate([h0, h1], axis=1)
    sc = jnp.sum(h * wa[...], axis=1, keepdims=True) + ba[...]
    e = jnp.exp(jnp.tanh(sc))
    rows = i * 128 + lax.broadcasted_iota(jnp.int32, (128, 1), 0)
    e = jnp.where(rows < N, e, 0.0)
    onehot = (lax.broadcasted_iota(jnp.int32, (G, 128), 0)
              == batch[0]).astype(_f32)

    @pl.when(i == 0)
    def _():
        p_acc[...] = jnp.zeros_like(p_acc)
        s_acc[0, 0] = 0.0

    p_acc[...] += jnp.dot(onehot, h * e, preferred_element_type=_f32)
    s_acc[0, 0] += jnp.sum(e)

    @pl.when(i == NP // 128 - 1)
    def _():
        pooled = p_acc[...] / s_acc[0, 0]
        o = jnp.dot(pooled, wp[...], preferred_element_type=_f32) + bp[...]
        nrm = jnp.sqrt(jnp.sum(o * o, axis=1, keepdims=True))
        out[...] = o / jnp.maximum(nrm, 1e-12)


def _tc_first(deg3, x_pad, w):
    return pl.pallas_call(
        _first_body,
        grid=(NP // 128, 2, 2),
        in_specs=[
            pl.BlockSpec((2, 128, 128), lambda i, j, k: (0, i, 0)),
            pl.BlockSpec((128, 128), lambda i, j, k: (i, k)),
            pl.BlockSpec((128, 128), lambda i, j, k: (k, j)),
        ],
        out_specs=pl.BlockSpec((1, 128, 128), lambda i, j, k: (j, i, 0)),
        out_shape=jax.ShapeDtypeStruct((2, NP, 128), _f32),
        scratch_shapes=[pltpu.VMEM((128, 128), _f32)],
    )(deg3, x_pad, w)


def _tc_mid(deg3, agg3, y3, w, b2):
    return pl.pallas_call(
        _mid_body,
        grid=(NP // 128, 2, 2),
        in_specs=[
            pl.BlockSpec((2, 128, 128), lambda i, j, k: (0, i, 0)),
            pl.BlockSpec((1, 128, 128), lambda i, j, k: (k, i, 0)),
            pl.BlockSpec((1, 128, 128), lambda i, j, k: (k, i, 0)),
            pl.BlockSpec((128, 128), lambda i, j, k: (k, j)),
            pl.BlockSpec((1, 1, 128), lambda i, j, k: (k, 0, 0)),
        ],
        out_specs=pl.BlockSpec((1, 128, 128), lambda i, j, k: (j, i, 0)),
        out_shape=jax.ShapeDtypeStruct((2, NP, 128), _f32),
        scratch_shapes=[pltpu.VMEM((128, 128), _f32)],
    )(deg3, agg3, y3, w, b2)


def _tc_final(deg3, agg3, y3, b2, wa_row, ba2, wp, bp2, batch2):
    return pl.pallas_call(
        _final_body,
        grid=(NP // 128,),
        in_specs=[
            pl.BlockSpec((2, 128, 128), lambda i: (0, i, 0)),
            pl.BlockSpec((2, 128, 128), lambda i: (0, i, 0)),
            pl.BlockSpec((2, 128, 128), lambda i: (0, i, 0)),
            pl.BlockSpec((2, 1, 128), lambda i: (0, 0, 0)),
            pl.BlockSpec((1, 256), lambda i: (0, 0)),
            pl.BlockSpec((1, 1), lambda i: (0, 0)),
            pl.BlockSpec((256, 256), lambda i: (0, 0)),
            pl.BlockSpec((1, 256), lambda i: (0, 0)),
            pl.BlockSpec((1, 1, 128), lambda i: (i, 0, 0)),
        ],
        out_specs=pl.BlockSpec((G, 256), lambda i: (0, 0)),
        out_shape=jax.ShapeDtypeStruct((G, 256), _f32),
        scratch_shapes=[pltpu.VMEM((G, 256), _f32),
                        pltpu.SMEM((1, 1), _f32)],
    )(deg3, agg3, y3, b2, wa_row, ba2, wp, bp2, batch2)


def kernel(x, edge_index, batch, W1, b1, W2, b2, W3, b3, Wa, ba, Wp, bp):
    src = edge_index[0].astype(jnp.int32)
    dst = edge_index[1].astype(jnp.int32)
    pad = jnp.full((E2 - E,), NP - 1, dtype=jnp.int32)
    src_p = jnp.concatenate([src, pad])
    dst_p = jnp.concatenate([dst, pad])
    srcp = jnp.stack([src_p, src_p + NP]).reshape(NC, EROWS, 128)
    dst2 = dst_p.reshape(EROWS, 128)

    x_pad = jnp.zeros((NP, D), _f32).at[:N].set(x)
    batch2 = jnp.zeros((NP,), jnp.int32).at[:N].set(
        batch.astype(jnp.int32)).reshape(NP // 128, 128)

    zrows = jnp.zeros((NROWS_PS, DH), _f32)
    ones_tab = jnp.ones((128, 128), _f32)

    deg = _sc_deg(dst2, ones_tab, zrows)
    deg3 = deg.reshape(NC, NP, 128)

    y1 = _tc_first(deg3, x_pad, W1)
    agg1 = _sc_agg(y1.reshape(NC * NP, DH), srcp, dst2, zrows)
    y2 = _tc_mid(deg3, agg1.reshape(NC, NP, DH), y1, W2, b1.reshape(2, 1, 128))
    agg2 = _sc_agg(y2.reshape(NC * NP, DH), srcp, dst2, zrows)
    y3 = _tc_mid(deg3, agg2.reshape(NC, NP, DH), y2, W3, b2.reshape(2, 1, 128))
    agg3 = _sc_agg(y3.reshape(NC * NP, DH), srcp, dst2, zrows)

    out = _tc_final(
        deg3, agg3.reshape(NC, NP, DH), y3, b3.reshape(2, 1, 128),
        Wa.reshape(1, D), ba.reshape(1, 1), Wp, bp.reshape(1, D),
        batch2.reshape(NP // 128, 1, 128))
    return out
